# Initial kernel scaffold; baseline (speedup 1.0000x reference)
import jax
import jax.numpy as jnp
from jax.experimental import pallas as pl


def kernel(queries, keys, vals):
    raise NotImplementedError("write your pallas kernel here")


# speedup vs baseline: 3.1248x; 3.1248x over previous
import functools

import jax
import jax.numpy as jnp
from jax.experimental import pallas as pl

NUM_HEADS = 8
HEAD_DIM = 64
STORAGE_SIZE = 1024
CHUNK_SIZE = 64
RETRIEVAL_K = 8
SEQ_LEN = 512
BATCH = 4


def _mean_kernel(k_ref, v_ref, ck_ref, cv_ref):
    def seq_mean(x):
        acc = x[:, :, 0, :]
        for c in range(1, x.shape[2]):
            acc = acc + x[:, :, c, :]
        return acc * (1.0 / x.shape[2])

    ck_ref[...] = seq_mean(k_ref[...])
    cv_ref[...] = seq_mean(v_ref[...])


def _topk_gather_kernel(q_ref, ck_ref, cv_ref, ok_ref, ov_ref, *, S, N, K):
    q = q_ref[0, 0]
    ck = ck_ref[0]
    cv = cv_ref[0]
    scores = jax.lax.dot_general(
        q, ck, (((1,), (1,)), ((), ())), preferred_element_type=jnp.float32
    )
    iota = jax.lax.broadcasted_iota(jnp.int32, (S, N), 1)
    for k in range(K):
        m = jnp.max(scores, axis=1, keepdims=True)
        sel = jnp.where(scores >= m, iota, N)
        arg = jnp.min(sel, axis=1, keepdims=True)
        onehot = (iota == arg).astype(jnp.float32)
        ok_ref[0, :, k, :] = jax.lax.dot_general(
            onehot, ck, (((1,), (0,)), ((), ())),
            preferred_element_type=jnp.float32,
            precision=jax.lax.Precision.HIGHEST,
        )
        ov_ref[0, :, k, :] = jax.lax.dot_general(
            onehot, cv, (((1,), (0,)), ((), ())),
            preferred_element_type=jnp.float32,
            precision=jax.lax.Precision.HIGHEST,
        )
        scores = jnp.where(iota == arg, -jnp.inf, scores)


def kernel(queries, keys, vals):
    S, B, D = queries.shape
    H, N, C, Dh = keys.shape
    K = RETRIEVAL_K

    NB = 128
    chunk_keys, chunk_vals = pl.pallas_call(
        _mean_kernel,
        grid=(H, N // NB),
        in_specs=[
            pl.BlockSpec((1, NB, C, Dh), lambda h, n: (h, n, 0, 0)),
            pl.BlockSpec((1, NB, C, Dh), lambda h, n: (h, n, 0, 0)),
        ],
        out_specs=[
            pl.BlockSpec((1, NB, Dh), lambda h, n: (h, n, 0)),
            pl.BlockSpec((1, NB, Dh), lambda h, n: (h, n, 0)),
        ],
        out_shape=[
            jax.ShapeDtypeStruct((H, N, Dh), jnp.float32),
            jax.ShapeDtypeStruct((H, N, Dh), jnp.float32),
        ],
    )(keys, vals)

    q4 = queries.reshape(S, B, H, Dh).transpose(1, 2, 0, 3)

    out_keys, out_vals = pl.pallas_call(
        functools.partial(_topk_gather_kernel, S=S, N=N, K=K),
        grid=(B, H),
        in_specs=[
            pl.BlockSpec((1, 1, S, Dh), lambda b, h: (b, h, 0, 0)),
            pl.BlockSpec((1, N, Dh), lambda b, h: (h, 0, 0)),
            pl.BlockSpec((1, N, Dh), lambda b, h: (h, 0, 0)),
        ],
        out_specs=[
            pl.BlockSpec((1, S, K, Dh), lambda b, h: (b * NUM_HEADS + h, 0, 0, 0)),
            pl.BlockSpec((1, S, K, Dh), lambda b, h: (b * NUM_HEADS + h, 0, 0, 0)),
        ],
        out_shape=[
            jax.ShapeDtypeStruct((B * H, S, K, Dh), jnp.float32),
            jax.ShapeDtypeStruct((B * H, S, K, Dh), jnp.float32),
        ],
    )(q4, chunk_keys, chunk_vals)

    return (out_keys, out_vals)


# speedup vs baseline: 4.2555x; 1.3618x over previous
import functools

import jax
import jax.numpy as jnp
from jax import lax
from jax.experimental import pallas as pl
from jax.experimental.pallas import tpu as pltpu, tpu_sc as plsc

NUM_HEADS = 8
HEAD_DIM = 64
STORAGE_SIZE = 1024
CHUNK_SIZE = 64
RETRIEVAL_K = 8
SEQ_LEN = 512
BATCH = 4


def _mean_kernel(k_ref, v_ref, ckv_ref):
    def seq_mean(x):
        acc = x[:, :, 0, :]
        for c in range(1, x.shape[2]):
            acc = acc + x[:, :, c, :]
        return acc * (1.0 / x.shape[2])

    ckv_ref[...] = jnp.concatenate(
        [seq_mean(k_ref[...]), seq_mean(v_ref[...])], axis=-1
    )


def _topk_kernel(q_ref, ckv_ref, idx_ref, *, S, N, K, Dh):
    h = pl.program_id(1)
    q = q_ref[0, 0]
    ck = ckv_ref[0][:, :Dh]
    scores = jax.lax.dot_general(
        q, ck, (((1,), (1,)), ((), ())), preferred_element_type=jnp.float32
    )
    iota = jax.lax.broadcasted_iota(jnp.int32, (S, N), 1)
    cols = []
    for k in range(K):
        m = jnp.max(scores, axis=1, keepdims=True)
        sel = jnp.where(scores == m, iota, N)
        arg = jnp.min(sel, axis=1, keepdims=True)
        cols.append(arg)
        if k < K - 1:
            scores = jnp.where(iota == arg, -jnp.inf, scores)
    idx = jnp.concatenate(cols, axis=1)
    idx_ref[0] = idx + h * N


def _make_sc_gather(R, W, n_workers, chunk):
    rows_per_w = R // n_workers
    n_chunks = rows_per_w // chunk
    mesh = plsc.VectorSubcoreMesh(core_axis_name="c", subcore_axis_name="s")

    @functools.partial(
        pl.kernel,
        mesh=mesh,
        out_type=jax.ShapeDtypeStruct((R, W), jnp.float32),
        scratch_types=[
            pltpu.VMEM((chunk,), jnp.int32),
            pltpu.VMEM((chunk, W), jnp.float32),
            pltpu.SemaphoreType.DMA,
        ],
    )
    def gather_k(table_hbm, idx_hbm, out_hbm, idx_v, rows_v, sem):
        wid = lax.axis_index("s") * 2 + lax.axis_index("c")
        base = wid * rows_per_w

        @pl.loop(0, n_chunks)
        def _chunk(j):
            off = base + j * chunk
            pltpu.sync_copy(idx_hbm.at[pl.ds(off, chunk)], idx_v)
            pltpu.async_copy(table_hbm.at[idx_v], rows_v, sem).wait()
            pltpu.sync_copy(rows_v, out_hbm.at[pl.ds(off, chunk)])

    return gather_k


def kernel(queries, keys, vals):
    S, B, D = queries.shape
    H, N, C, Dh = keys.shape
    K = RETRIEVAL_K
    W = 2 * Dh

    NB = 128
    ckv = pl.pallas_call(
        _mean_kernel,
        grid=(H, N // NB),
        in_specs=[
            pl.BlockSpec((1, NB, C, Dh), lambda h, n: (h, n, 0, 0)),
            pl.BlockSpec((1, NB, C, Dh), lambda h, n: (h, n, 0, 0)),
        ],
        out_specs=pl.BlockSpec((1, NB, W), lambda h, n: (h, n, 0)),
        out_shape=jax.ShapeDtypeStruct((H, N, W), jnp.float32),
    )(keys, vals)

    q4 = queries.reshape(S, B, H, Dh).transpose(1, 2, 0, 3)

    top_idx = pl.pallas_call(
        functools.partial(_topk_kernel, S=S, N=N, K=K, Dh=Dh),
        grid=(B, H),
        in_specs=[
            pl.BlockSpec((1, 1, S, Dh), lambda b, h: (b, h, 0, 0)),
            pl.BlockSpec((1, N, W), lambda b, h: (h, 0, 0)),
        ],
        out_specs=pl.BlockSpec((1, S, K), lambda b, h: (b * NUM_HEADS + h, 0, 0)),
        out_shape=jax.ShapeDtypeStruct((B * H, S, K), jnp.int32),
    )(q4, ckv)

    R = B * H * S * K
    gather = _make_sc_gather(R, W, n_workers=32, chunk=128)
    comb = gather(ckv.reshape(H * N, W), top_idx.reshape(R))
    comb = comb.reshape(B * H, S, K, W)
    return (comb[..., :Dh], comb[..., Dh:])


# speedup vs baseline: 5.6872x; 1.3364x over previous
import functools

import jax
import jax.numpy as jnp
from jax import lax
from jax.experimental import pallas as pl
from jax.experimental.pallas import tpu as pltpu, tpu_sc as plsc

NUM_HEADS = 8
HEAD_DIM = 64
STORAGE_SIZE = 1024
CHUNK_SIZE = 64
RETRIEVAL_K = 8
SEQ_LEN = 512
BATCH = 4


def _mean_kernel(k_ref, v_ref, ckv_ref, *, C, Dh):
    cp = pl.program_id(2)
    n_cp = C // 2
    kb = k_ref[0]
    vb = v_ref[0]

    @pl.when(cp == 0)
    def _init():
        ckv_ref[0, :, :Dh] = kb[:, :Dh] + kb[:, Dh:]
        ckv_ref[0, :, Dh:] = vb[:, :Dh] + vb[:, Dh:]

    @pl.when(jnp.logical_and(cp > 0, cp < n_cp - 1))
    def _acc():
        ckv_ref[0, :, :Dh] = (ckv_ref[0, :, :Dh] + kb[:, :Dh]) + kb[:, Dh:]
        ckv_ref[0, :, Dh:] = (ckv_ref[0, :, Dh:] + vb[:, :Dh]) + vb[:, Dh:]

    @pl.when(cp == n_cp - 1)
    def _fin():
        ckv_ref[0, :, :Dh] = ((ckv_ref[0, :, :Dh] + kb[:, :Dh]) + kb[:, Dh:]) * (1.0 / C)
        ckv_ref[0, :, Dh:] = ((ckv_ref[0, :, Dh:] + vb[:, :Dh]) + vb[:, Dh:]) * (1.0 / C)


def _topk_kernel(q_ref, ckv_ref, idx_ref, *, S, N, K, Dh):
    h = pl.program_id(1)
    q = q_ref[0, 0]
    ck = ckv_ref[0][:, :Dh]
    scores = jax.lax.dot_general(
        q, ck, (((1,), (1,)), ((), ())), preferred_element_type=jnp.float32
    )
    iota = jax.lax.broadcasted_iota(jnp.int32, (S, N), 1)
    cols = []
    for k in range(K):
        m = jnp.max(scores, axis=1, keepdims=True)
        sel = jnp.where(scores == m, iota, N)
        arg = jnp.min(sel, axis=1, keepdims=True)
        cols.append(arg)
        if k < K - 1:
            scores = jnp.where(iota == arg, -jnp.inf, scores)
    idx = jnp.concatenate(cols, axis=1)
    idx_ref[0] = idx + h * N


def _make_sc_gather(R, W, n_workers, chunk):
    rows_per_w = R // n_workers
    n_chunks = rows_per_w // chunk
    mesh = plsc.VectorSubcoreMesh(core_axis_name="c", subcore_axis_name="s")

    @functools.partial(
        pl.kernel,
        mesh=mesh,
        out_type=jax.ShapeDtypeStruct((R, W), jnp.float32),
        scratch_types=[
            pltpu.VMEM((chunk,), jnp.int32),
            pltpu.VMEM((chunk, W), jnp.float32),
            pltpu.SemaphoreType.DMA,
        ],
    )
    def gather_k(table_hbm, idx_hbm, out_hbm, idx_v, rows_v, sem):
        wid = lax.axis_index("s") * 2 + lax.axis_index("c")
        base = wid * rows_per_w

        @pl.loop(0, n_chunks)
        def _chunk(j):
            off = base + j * chunk
            pltpu.sync_copy(idx_hbm.at[pl.ds(off, chunk)], idx_v)
            pltpu.async_copy(table_hbm.at[idx_v], rows_v, sem).wait()
            pltpu.sync_copy(rows_v, out_hbm.at[pl.ds(off, chunk)])

    return gather_k


def kernel(queries, keys, vals):
    S, B, D = queries.shape
    H, N, C, Dh = keys.shape
    K = RETRIEVAL_K
    W = 2 * Dh

    NB = 512
    keys_l = keys.reshape(H, N, C * Dh)
    vals_l = vals.reshape(H, N, C * Dh)
    ckv = pl.pallas_call(
        functools.partial(_mean_kernel, C=C, Dh=Dh),
        grid=(H, N // NB, C // 2),
        in_specs=[
            pl.BlockSpec((1, NB, W), lambda h, n, cp: (h, n, cp)),
            pl.BlockSpec((1, NB, W), lambda h, n, cp: (h, n, cp)),
        ],
        out_specs=pl.BlockSpec((1, NB, W), lambda h, n, cp: (h, n, 0)),
        out_shape=jax.ShapeDtypeStruct((H, N, W), jnp.float32),
    )(keys_l, vals_l)

    q4 = queries.reshape(S, B, H, Dh).transpose(1, 2, 0, 3)

    top_idx = pl.pallas_call(
        functools.partial(_topk_kernel, S=S, N=N, K=K, Dh=Dh),
        grid=(B, H),
        in_specs=[
            pl.BlockSpec((1, 1, S, Dh), lambda b, h: (b, h, 0, 0)),
            pl.BlockSpec((1, N, W), lambda b, h: (h, 0, 0)),
        ],
        out_specs=pl.BlockSpec((1, S, K), lambda b, h: (b * NUM_HEADS + h, 0, 0)),
        out_shape=jax.ShapeDtypeStruct((B * H, S, K), jnp.int32),
    )(q4, ckv)

    R = B * H * S * K
    gather = _make_sc_gather(R, W, n_workers=32, chunk=128)
    comb = gather(ckv.reshape(H * N, W), top_idx.reshape(R))
    comb = comb.reshape(B * H, S, K, W)
    return (comb[..., :Dh], comb[..., Dh:])


# speedup vs baseline: 7.3170x; 1.2866x over previous
import functools

import jax
import jax.numpy as jnp
from jax import lax
from jax.experimental import pallas as pl
from jax.experimental.pallas import tpu as pltpu, tpu_sc as plsc

NUM_HEADS = 8
HEAD_DIM = 64
STORAGE_SIZE = 1024
CHUNK_SIZE = 64
RETRIEVAL_K = 8
SEQ_LEN = 512
BATCH = 4


def _mean_kernel(k_ref, v_ref, ckv_ref, *, C, Dh):
    def seq_mean(x):
        acc = x[:, 0:Dh] + x[:, Dh : 2 * Dh]
        for cp in range(1, C // 2):
            off = cp * 2 * Dh
            acc = (acc + x[:, off : off + Dh]) + x[:, off + Dh : off + 2 * Dh]
        return acc * (1.0 / C)

    ckv_ref[0] = jnp.concatenate(
        [seq_mean(k_ref[0]), seq_mean(v_ref[0])], axis=-1
    )


def _topk_kernel(q_ref, ckv_ref, idx_ref, *, S, N, K, Dh):
    h = pl.program_id(1)
    q = q_ref[0, 0]
    ck = ckv_ref[0][:, :Dh]
    scores = jax.lax.dot_general(
        q, ck, (((1,), (1,)), ((), ())), preferred_element_type=jnp.float32
    )
    iota = jax.lax.broadcasted_iota(jnp.int32, (S, N), 1)
    cols = []
    for k in range(K):
        m = jnp.max(scores, axis=1, keepdims=True)
        sel = jnp.where(scores == m, iota, N)
        arg = jnp.min(sel, axis=1, keepdims=True)
        cols.append(arg)
        if k < K - 1:
            scores = jnp.where(iota == arg, -jnp.inf, scores)
    idx = jnp.concatenate(cols, axis=1)
    idx_ref[0] = idx + h * N


def _make_sc_gather(R, W, n_workers, chunk):
    rows_per_w = R // n_workers
    n_chunks = rows_per_w // chunk
    mesh = plsc.VectorSubcoreMesh(core_axis_name="c", subcore_axis_name="s")

    @functools.partial(
        pl.kernel,
        mesh=mesh,
        out_type=jax.ShapeDtypeStruct((R, W), jnp.float32),
        scratch_types=[
            pltpu.VMEM((chunk,), jnp.int32),
            pltpu.VMEM((chunk, W), jnp.float32),
            pltpu.SemaphoreType.DMA,
        ],
    )
    def gather_k(table_hbm, idx_hbm, out_hbm, idx_v, rows_v, sem):
        wid = lax.axis_index("s") * 2 + lax.axis_index("c")
        base = wid * rows_per_w

        @pl.loop(0, n_chunks)
        def _chunk(j):
            off = base + j * chunk
            pltpu.sync_copy(idx_hbm.at[pl.ds(off, chunk)], idx_v)
            pltpu.async_copy(table_hbm.at[idx_v], rows_v, sem).wait()
            pltpu.sync_copy(rows_v, out_hbm.at[pl.ds(off, chunk)])

    return gather_k


def kernel(queries, keys, vals):
    S, B, D = queries.shape
    H, N, C, Dh = keys.shape
    K = RETRIEVAL_K
    W = 2 * Dh

    NB = 128
    keys_l = keys.reshape(H, N, C * Dh)
    vals_l = vals.reshape(H, N, C * Dh)
    ckv = pl.pallas_call(
        functools.partial(_mean_kernel, C=C, Dh=Dh),
        grid=(H, N // NB),
        in_specs=[
            pl.BlockSpec((1, NB, C * Dh), lambda h, n: (h, n, 0)),
            pl.BlockSpec((1, NB, C * Dh), lambda h, n: (h, n, 0)),
        ],
        out_specs=pl.BlockSpec((1, NB, W), lambda h, n: (h, n, 0)),
        out_shape=jax.ShapeDtypeStruct((H, N, W), jnp.float32),
    )(keys_l, vals_l)

    q4 = queries.reshape(S, B, H, Dh).transpose(1, 2, 0, 3)

    top_idx = pl.pallas_call(
        functools.partial(_topk_kernel, S=S, N=N, K=K, Dh=Dh),
        grid=(B, H),
        in_specs=[
            pl.BlockSpec((1, 1, S, Dh), lambda b, h: (b, h, 0, 0)),
            pl.BlockSpec((1, N, W), lambda b, h: (h, 0, 0)),
        ],
        out_specs=pl.BlockSpec((1, S, K), lambda b, h: (b * NUM_HEADS + h, 0, 0)),
        out_shape=jax.ShapeDtypeStruct((B * H, S, K), jnp.int32),
    )(q4, ckv)

    R = B * H * S * K
    gather = _make_sc_gather(R, W, n_workers=32, chunk=128)
    comb = gather(ckv.reshape(H * N, W), top_idx.reshape(R))
    comb = comb.reshape(B * H, S, K, W)
    return (comb[..., :Dh], comb[..., Dh:])
